# baseline (speedup 1.0000x reference)
import functools

import jax
import jax.numpy as jnp
from jax import lax
from jax.experimental import pallas as pl
from jax.experimental.pallas import tpu as pltpu
from jax.experimental.pallas import tpu_sc as plsc

_NC = 2
_NS = 16
_NW = _NC * _NS
_L = 16


def _shuffle_xor(x, m):
    perm = lax.iota(jnp.int32, _L) ^ m
    return jnp.take_along_axis(x, perm, axis=0)


def _tree_sum(terms):
    while len(terms) > 1:
        nxt = [terms[i] + terms[i + 1] for i in range(0, len(terms) - 1, 2)]
        if len(terms) % 2:
            nxt.append(terms[-1])
        terms = nxt
    return terms[0]


def _butterfly_reduce(vecs):
    iota = lax.iota(jnp.int32, _L)
    s = 1
    while len(vecs) > 1:
        m = (iota & s) != 0
        nxt = []
        for i in range(0, len(vecs), 2):
            a, b = vecs[i], vecs[i + 1]
            x = jnp.where(m, b, a)
            y = jnp.where(m, a, b)
            nxt.append(x + _shuffle_xor(y, s))
        vecs = nxt
        s *= 2
    return vecs[0]


def _build_sc_call(N, D, Td, P, R, C):
    rows_per_w = N // _NW
    CH = rows_per_w // C
    KD = D // _L
    KT = Td // _L
    G = C * R
    assert G == 128
    IW = rows_per_w * R // 128
    PAIR = _L // R

    mesh = plsc.VectorSubcoreMesh(core_axis_name="c", subcore_axis_name="s")

    @functools.partial(
        pl.kernel,
        mesh=mesh,
        out_type=jax.ShapeDtypeStruct((N, D + Td), jnp.float32),
        compiler_params=pltpu.CompilerParams(needs_layout_passes=False),
        scratch_types=[
            pltpu.VMEM((IW, 128), jnp.int32),
            pltpu.VMEM((IW, 128), jnp.int32),
            pltpu.VMEM((IW, 128), jnp.int32),
            pltpu.VMEM((IW, 128), jnp.int32),
            pltpu.VMEM((2, C, D), jnp.float32),
            pltpu.VMEM((2, G, D), jnp.float32),
            pltpu.VMEM((2, G, Td), jnp.float32),
            pltpu.VMEM((G,), jnp.float32),
            pltpu.VMEM((G,), jnp.float32),
            pltpu.VMEM((rows_per_w,), jnp.int32),
            pltpu.VMEM((2, C, Td), jnp.float32),
            pltpu.SemaphoreType.DMA((2,)),
        ],
    )
    def sc_fn(inp_hbm, temb_hbm, pos_hbm, job_hbm, tit_hbm, ey_hbm, out_hbm,
              pos_v, job_v, fidx_v, tidx_v, focal_v, ref_v, trow_v,
              s_v, sim_v, ey_v, out_v, sem_in):
        wid = lax.axis_index("s") * _NC + lax.axis_index("c")
        base0 = wid * rows_per_w

        pltpu.sync_copy(pos_hbm.at[pl.ds(wid * IW, IW)], pos_v)
        pltpu.sync_copy(job_hbm.at[pl.ds(wid * IW, IW)], job_v)
        pltpu.sync_copy(tit_hbm.at[pl.ds(wid * IW, IW)], tidx_v)
        pltpu.sync_copy(ey_hbm.at[pl.ds(base0, rows_per_w)], ey_v)

        def flat_body(row, carry):
            for cc in range(128 // _L):
                f = (pos_v[row, pl.ds(cc * _L, _L)] * P
                     + job_v[row, pl.ds(cc * _L, _L)])
                fidx_v[row, pl.ds(cc * _L, _L)] = f
            return carry
        lax.fori_loop(0, IW, flat_body, 0)

        def issue(ci, p):
            base = base0 + ci * C
            pltpu.async_copy(inp_hbm.at[fidx_v.at[ci]], ref_v.at[p],
                             sem_in.at[p])
            pltpu.async_copy(temb_hbm.at[tidx_v.at[ci]], trow_v.at[p],
                             sem_in.at[p])
            pltpu.async_copy(inp_hbm.at[pl.ds(base, C)], focal_v.at[p],
                             sem_in.at[p])

        def drain(ci, p):
            base = base0 + ci * C
            pltpu.make_async_copy(inp_hbm.at[fidx_v.at[ci]], ref_v.at[p],
                                  sem_in.at[p]).wait()
            pltpu.make_async_copy(temb_hbm.at[tidx_v.at[ci]], trow_v.at[p],
                                  sem_in.at[p]).wait()
            pltpu.make_async_copy(inp_hbm.at[pl.ds(base, C)], focal_v.at[p],
                                  sem_in.at[p]).wait()

        issue(0, 0)
        iota = lax.iota(jnp.int32, _L)

        def chunk_body(ci, carry):
            p = ci & 1
            base = base0 + ci * C

            @pl.when(ci + 1 < CH)
            def _():
                issue(ci + 1, 1 - p)

            drain(ci, p)

            @plsc.parallel_loop(0, C // PAIR, unroll=2)
            def dot_body(m):
                accs = []
                for dn in range(PAIR):
                    n = m * PAIR + dn
                    j0 = n * R
                    fv = [focal_v[p, n, pl.ds(k * _L, _L)]
                          for k in range(KD)]
                    for r in range(R):
                        prods = [fv[k] * ref_v[p, j0 + r, pl.ds(k * _L, _L)]
                                 for k in range(KD)]
                        accs.append(_tree_sum(prods))
                s_v[pl.ds(m * _L, _L)] = _butterfly_reduce(accs)

            for m in range(C // PAIR):
                e = jnp.exp(s_v[pl.ds(m * _L, _L)])
                den = e
                mm = 1
                while mm < R:
                    den = den + _shuffle_xor(den, mm)
                    mm *= 2
                inv = 1.0 / den
                ey_idx = (jnp.full((_L,), ci * C + m * PAIR, jnp.int32)
                          + lax.shift_right_logical(iota, 3))
                keep = plsc.load_gather(ey_v, [ey_idx]) != 0
                zero = jnp.zeros((_L,), jnp.float32)
                sim_v[pl.ds(m * _L, _L)] = jnp.where(keep, e * inv, zero)

            @plsc.parallel_loop(0, C, unroll=4)
            def out_body(n):
                j0 = n * R
                ws = [plsc.load_gather(
                          sim_v, [jnp.full((_L,), r, jnp.int32) + n * R])
                      for r in range(R)]
                for k in range(KT):
                    acc = _tree_sum(
                        [ws[r] * trow_v[p, j0 + r, pl.ds(k * _L, _L)]
                         for r in range(R)])
                    out_v[p, n, pl.ds(k * _L, _L)] = acc

            pltpu.sync_copy(focal_v.at[p],
                            out_hbm.at[pl.ds(base, C), pl.ds(0, D)])
            pltpu.sync_copy(out_v.at[p],
                            out_hbm.at[pl.ds(base, C), pl.ds(D, Td)])
            return carry

        lax.fori_loop(0, CH, chunk_body, 0)

    return sc_fn


def kernel(title_emb_mat, emp_ids, end_yrs, batch_label, inputs,
           ref_batch_pos, ref_job_idx, ref_title_idx):
    B, P, D = inputs.shape
    T, Td = title_emb_mat.shape
    R = ref_batch_pos.shape[-1]
    N = B * P

    inp_flat = inputs.reshape(N, D)
    pos_rows = ref_batch_pos.astype(jnp.int32).reshape(-1, 128)
    job_rows = ref_job_idx.astype(jnp.int32).reshape(-1, 128)
    tit_rows = ref_title_idx.astype(jnp.int32).reshape(-1, 128)
    ey_flat = end_yrs.astype(jnp.int32).reshape(-1)

    fn = _build_sc_call(N, D, Td, P, R, C=128 // R)
    return fn(inp_flat, title_emb_mat, pos_rows, job_rows, tit_rows, ey_flat)


# speedup vs baseline: 1.0276x; 1.0276x over previous
import functools

import jax
import jax.numpy as jnp
from jax import lax
from jax.experimental import pallas as pl
from jax.experimental.pallas import tpu as pltpu
from jax.experimental.pallas import tpu_sc as plsc

_NC = 2
_NS = 16
_NW = _NC * _NS
_L = 16


def _shuffle_xor(x, m):
    perm = lax.iota(jnp.int32, _L) ^ m
    return jnp.take_along_axis(x, perm, axis=0)


def _tree_sum(terms):
    while len(terms) > 1:
        nxt = [terms[i] + terms[i + 1] for i in range(0, len(terms) - 1, 2)]
        if len(terms) % 2:
            nxt.append(terms[-1])
        terms = nxt
    return terms[0]


def _butterfly_reduce(vecs):
    iota = lax.iota(jnp.int32, _L)
    s = 1
    while len(vecs) > 1:
        m = (iota & s) != 0
        nxt = []
        for i in range(0, len(vecs), 2):
            a, b = vecs[i], vecs[i + 1]
            x = jnp.where(m, b, a)
            y = jnp.where(m, a, b)
            nxt.append(x + _shuffle_xor(y, s))
        vecs = nxt
        s *= 2
    return vecs[0]


def _build_sc_call(N, D, Td, P, R, C):
    rows_per_w = N // _NW
    CH = rows_per_w // C
    KD = D // _L
    KT = Td // _L
    G = C * R
    assert G == 128
    IW = rows_per_w * R // 128
    PAIR = _L // R

    mesh = plsc.VectorSubcoreMesh(core_axis_name="c", subcore_axis_name="s")

    @functools.partial(
        pl.kernel,
        mesh=mesh,
        out_type=jax.ShapeDtypeStruct((N, D + Td), jnp.float32),
        compiler_params=pltpu.CompilerParams(needs_layout_passes=False),
        scratch_types=[
            pltpu.VMEM((IW, 128), jnp.int32),
            pltpu.VMEM((IW, 128), jnp.int32),
            pltpu.VMEM((IW, 128), jnp.int32),
            pltpu.VMEM((IW, 128), jnp.int32),
            pltpu.VMEM((3, C, D), jnp.float32),
            pltpu.VMEM((2, G, D), jnp.float32),
            pltpu.VMEM((2, G, Td), jnp.float32),
            pltpu.VMEM((G,), jnp.float32),
            pltpu.VMEM((G,), jnp.float32),
            pltpu.VMEM((rows_per_w,), jnp.int32),
            pltpu.VMEM((3, C, Td), jnp.float32),
            pltpu.SemaphoreType.DMA((2,)),
            pltpu.SemaphoreType.DMA((3,)),
        ],
    )
    def sc_fn(inp_hbm, temb_hbm, pos_hbm, job_hbm, tit_hbm, ey_hbm, out_hbm,
              pos_v, job_v, fidx_v, tidx_v, focal_v, ref_v, trow_v,
              s_v, sim_v, ey_v, out_v, sem_in, sem_out):
        wid = lax.axis_index("s") * _NC + lax.axis_index("c")
        base0 = wid * rows_per_w

        pltpu.sync_copy(pos_hbm.at[pl.ds(wid * IW, IW)], pos_v)
        pltpu.sync_copy(job_hbm.at[pl.ds(wid * IW, IW)], job_v)
        pltpu.sync_copy(tit_hbm.at[pl.ds(wid * IW, IW)], tidx_v)
        pltpu.sync_copy(ey_hbm.at[pl.ds(base0, rows_per_w)], ey_v)

        def flat_body(row, carry):
            for cc in range(128 // _L):
                f = (pos_v[row, pl.ds(cc * _L, _L)] * P
                     + job_v[row, pl.ds(cc * _L, _L)])
                fidx_v[row, pl.ds(cc * _L, _L)] = f
            return carry
        lax.fori_loop(0, IW, flat_body, 0)

        def issue(ci, p, pf):
            base = base0 + ci * C
            pltpu.async_copy(inp_hbm.at[fidx_v.at[ci]], ref_v.at[p],
                             sem_in.at[p])
            pltpu.async_copy(temb_hbm.at[tidx_v.at[ci]], trow_v.at[p],
                             sem_in.at[p])
            pltpu.async_copy(inp_hbm.at[pl.ds(base, C)], focal_v.at[pf],
                             sem_in.at[p])

        def drain(ci, p, pf):
            base = base0 + ci * C
            pltpu.make_async_copy(inp_hbm.at[fidx_v.at[ci]], ref_v.at[p],
                                  sem_in.at[p]).wait()
            pltpu.make_async_copy(temb_hbm.at[tidx_v.at[ci]], trow_v.at[p],
                                  sem_in.at[p]).wait()
            pltpu.make_async_copy(inp_hbm.at[pl.ds(base, C)], focal_v.at[pf],
                                  sem_in.at[p]).wait()

        def drain_out(cj):
            pfj = lax.rem(cj, 3)
            bj = base0 + cj * C
            pltpu.make_async_copy(
                focal_v.at[pfj], out_hbm.at[pl.ds(bj, C), pl.ds(0, D)],
                sem_out.at[pfj]).wait()
            pltpu.make_async_copy(
                out_v.at[pfj], out_hbm.at[pl.ds(bj, C), pl.ds(D, Td)],
                sem_out.at[pfj]).wait()

        issue(0, 0, 0)
        iota = lax.iota(jnp.int32, _L)

        def chunk_body(ci, carry):
            p = ci & 1
            pf = lax.rem(ci, 3)
            base = base0 + ci * C

            @pl.when(ci + 1 < CH)
            def _():
                @pl.when(ci >= 2)
                def _():
                    drain_out(ci - 2)
                issue(ci + 1, 1 - p, lax.rem(ci + 1, 3))

            drain(ci, p, pf)

            @plsc.parallel_loop(0, C // PAIR, unroll=2)
            def dot_body(m):
                accs = []
                for dn in range(PAIR):
                    n = m * PAIR + dn
                    j0 = n * R
                    fv = [focal_v[pf, n, pl.ds(k * _L, _L)]
                          for k in range(KD)]
                    for r in range(R):
                        prods = [fv[k] * ref_v[p, j0 + r, pl.ds(k * _L, _L)]
                                 for k in range(KD)]
                        accs.append(_tree_sum(prods))
                s_v[pl.ds(m * _L, _L)] = _butterfly_reduce(accs)

            for m in range(C // PAIR):
                e = jnp.exp(s_v[pl.ds(m * _L, _L)])
                den = e
                mm = 1
                while mm < R:
                    den = den + _shuffle_xor(den, mm)
                    mm *= 2
                inv = 1.0 / den
                ey_idx = (jnp.full((_L,), ci * C + m * PAIR, jnp.int32)
                          + lax.shift_right_logical(iota, 3))
                keep = plsc.load_gather(ey_v, [ey_idx]) != 0
                zero = jnp.zeros((_L,), jnp.float32)
                sim_v[pl.ds(m * _L, _L)] = jnp.where(keep, e * inv, zero)

            @plsc.parallel_loop(0, C, unroll=4)
            def out_body(n):
                j0 = n * R
                ws = [plsc.load_gather(
                          sim_v, [jnp.full((_L,), r, jnp.int32) + n * R])
                      for r in range(R)]
                for k in range(KT):
                    acc = _tree_sum(
                        [ws[r] * trow_v[p, j0 + r, pl.ds(k * _L, _L)]
                         for r in range(R)])
                    out_v[pf, n, pl.ds(k * _L, _L)] = acc

            pltpu.async_copy(focal_v.at[pf],
                             out_hbm.at[pl.ds(base, C), pl.ds(0, D)],
                             sem_out.at[pf])
            pltpu.async_copy(out_v.at[pf],
                             out_hbm.at[pl.ds(base, C), pl.ds(D, Td)],
                             sem_out.at[pf])
            return carry

        lax.fori_loop(0, CH, chunk_body, 0)
        drain_out(CH - 3)
        drain_out(CH - 2)
        drain_out(CH - 1)

    return sc_fn


def kernel(title_emb_mat, emp_ids, end_yrs, batch_label, inputs,
           ref_batch_pos, ref_job_idx, ref_title_idx):
    B, P, D = inputs.shape
    T, Td = title_emb_mat.shape
    R = ref_batch_pos.shape[-1]
    N = B * P

    inp_flat = inputs.reshape(N, D)
    pos_rows = ref_batch_pos.astype(jnp.int32).reshape(-1, 128)
    job_rows = ref_job_idx.astype(jnp.int32).reshape(-1, 128)
    tit_rows = ref_title_idx.astype(jnp.int32).reshape(-1, 128)
    ey_flat = end_yrs.astype(jnp.int32).reshape(-1)

    fn = _build_sc_call(N, D, Td, P, R, C=128 // R)
    return fn(inp_flat, title_emb_mat, pos_rows, job_rows, tit_rows, ey_flat)
